# Initial kernel scaffold; baseline (speedup 1.0000x reference)
import jax
import jax.numpy as jnp
from jax.experimental import pallas as pl


def kernel(feat0, feat1, feat2, feat3, w0, w1, w2, w3, b0, b1, b2, b3):
    raise NotImplementedError("write your pallas kernel here")


# speedup vs baseline: 1.1319x; 1.1319x over previous
import jax
import jax.numpy as jnp
from jax.experimental import pallas as pl
from jax.experimental.pallas import tpu as pltpu


def _chunk_expansion(wc, chunk, dtype):
    wf = 2 * wc
    cols = jnp.arange(2 * wf)
    src = (cols % wf) // 2
    e_row = (jnp.arange(wc)[:, None] == src[None, :]).astype(dtype)
    n = chunk // wc
    return jnp.kron(jnp.eye(n, dtype=dtype), e_row)


def _fused_fpn_kernel(*refs):
    xs = refs[0:4]
    ws = refs[4:8]
    bs = refs[8:12]
    es = refs[12:15]
    os_ = refs[15:19]

    acc = jnp.dot(ws[0][...], xs[0][0].astype(jnp.bfloat16),
                  preferred_element_type=jnp.float32)
    acc = acc + bs[0][...]
    os_[0][0] = acc.astype(os_[0].dtype)

    for k in (1, 2, 3):
        e = es[k - 1][...]
        chunk = e.shape[0]
        n_chunks = acc.shape[1] // chunk
        if n_chunks == 1:
            up = jnp.dot(acc, e, preferred_element_type=jnp.float32)
        else:
            parts = [
                jnp.dot(acc[:, i * chunk:(i + 1) * chunk], e,
                        preferred_element_type=jnp.float32)
                for i in range(n_chunks)
            ]
            up = jnp.concatenate(parts, axis=1)
        acc = jnp.dot(ws[k][...], xs[k][0].astype(jnp.bfloat16),
                      preferred_element_type=jnp.float32)
        acc = acc + bs[k][...] + up
        os_[k][0] = acc.astype(os_[k].dtype)


def kernel(feat0, feat1, feat2, feat3, w0, w1, w2, w3, b0, b1, b2, b3):
    feats = [feat0, feat1, feat2, feat3]
    ws = [w0, w1, w2, w3]
    bs = [b0, b1, b2, b3]

    B, _, H0, W0 = feat0.shape
    cout = w0.shape[0]
    dtype = feat0.dtype

    th0 = 4
    grid = (B, H0 // th0)

    xs = [f.reshape(B, f.shape[1], -1) for f in feats]
    ws_b = [w.astype(jnp.bfloat16) for w in ws]
    bs2 = [b.reshape(cout, 1).astype(jnp.float32) for b in bs]
    tss = [th0 * W0 * (4 ** k) for k in range(4)]

    es = []
    for k in (1, 2, 3):
        wc = W0 * (1 << (k - 1))
        chunk = wc * max(1, min(128, tss[k - 1]) // wc)
        es.append(_chunk_expansion(wc, chunk, jnp.float32))

    def spatial_spec(c, ts):
        return pl.BlockSpec((1, c, ts), lambda b, i: (b, 0, i))

    def const_spec(shape):
        return pl.BlockSpec(shape, lambda b, i: (0, 0))

    in_specs = (
        [spatial_spec(f.shape[1], ts) for f, ts in zip(feats, tss)]
        + [const_spec(w.shape) for w in ws_b]
        + [const_spec(b.shape) for b in bs2]
        + [const_spec(e.shape) for e in es]
    )
    out_specs = [spatial_spec(cout, ts) for ts in tss]
    out_shape = [jax.ShapeDtypeStruct((B, cout, ts * grid[1]), dtype)
                 for ts in tss]

    outs = pl.pallas_call(
        _fused_fpn_kernel,
        grid=grid,
        in_specs=in_specs,
        out_specs=out_specs,
        out_shape=out_shape,
        compiler_params=pltpu.CompilerParams(
            dimension_semantics=("parallel", "parallel"),
            vmem_limit_bytes=112 * 1024 * 1024),
    )(*xs, *ws_b, *bs2, *es)

    return [o.reshape(B, cout, H0 * (1 << k), W0 * (1 << k))
            for k, o in enumerate(outs)]
